# baseline (speedup 1.0000x reference)
import functools

import jax
import jax.numpy as jnp
from jax import lax
from jax.experimental import pallas as pl
from jax.experimental.pallas import tpu as pltpu
from jax.experimental.pallas import tpu_sc as plsc

R = 50000
RP = 50176
N_BOX = 5000
NBINS_CNT = 5008
C_OBJ = 151
C_REL = 51
CHUNK32 = RP // 32
CHUNK16 = RP // 16
NBINS = 2048
LAST_OFF = R - CHUNK32

_mesh = plsc.VectorSubcoreMesh(core_axis_name="c", subcore_axis_name="s")
_sc_params = pltpu.CompilerParams(needs_layout_passes=False)


@functools.partial(
    pl.kernel,
    out_type=jax.ShapeDtypeStruct((32, NBINS_CNT), jnp.int32),
    mesh=_mesh,
    scratch_types=[
        pltpu.VMEM((CHUNK32,), jnp.int32),
        pltpu.VMEM((NBINS_CNT,), jnp.int32),
    ],
    compiler_params=_sc_params,
)
def _count_kernel(sub_hbm, obj_hbm, out_hbm, idx_v, hist_v):
    cid = lax.axis_index("c")
    sid = lax.axis_index("s")
    w = cid * 16 + sid
    o = w * CHUNK32

    def zero_body(i, _):
        hist_v[pl.ds(i * 16, 16)] = jnp.zeros((16,), jnp.int32)
        return 0

    lax.fori_loop(0, NBINS_CNT // 16, zero_body, 0)

    def count_chunk(src):
        pltpu.sync_copy(src.at[pl.ds(o, CHUNK32)], idx_v)

        def body(i, _):
            v = idx_v[pl.ds(i * 16, 16)]
            c, m = plsc.scan_count(v)
            plsc.addupdate_scatter(hist_v, [v], c, mask=m)
            return 0

        lax.fori_loop(0, CHUNK32 // 16, body, 0)

    count_chunk(sub_hbm)
    count_chunk(obj_hbm)
    pltpu.sync_copy(hist_v, out_hbm.at[w])


def _make_pass(shift, first):
    n_in = 1 if first else 2

    def body(*refs):
        if first:
            key_in, dst_k, dst_p = refs[:3]
            src_p = None
        else:
            key_in, src_p, dst_k, dst_p = refs[:4]
        (keys_v, pays_v, pos2_v, hist_v, cur_v, grid_v, grid_s,
         sem_k, sem_p) = refs[n_in + 2:]
        cid = lax.axis_index("c")
        sid = lax.axis_index("s")
        o = sid * CHUNK16
        on_core0 = cid == 0

        @pl.when(on_core0)
        def _hist():
            pltpu.sync_copy(key_in.at[pl.ds(o, CHUNK16)], keys_v)
            if first:
                def ib(i, _):
                    pays_v[pl.ds(i * 16, 16)] = (
                        o + i * 16 + lax.iota(jnp.int32, 16))
                    return 0
                lax.fori_loop(0, CHUNK16 // 16, ib, 0)
            else:
                pltpu.sync_copy(src_p.at[pl.ds(o, CHUNK16)], pays_v)

            def zb(i, _):
                hist_v[pl.ds(i * 16, 16)] = jnp.zeros((16,), jnp.int32)
                return 0
            lax.fori_loop(0, NBINS // 16, zb, 0)

            def hb(i, _):
                v = keys_v[pl.ds(i * 16, 16)]
                d = lax.shift_right_logical(v, shift) & (NBINS - 1)
                c, m = plsc.scan_count(d)
                plsc.addupdate_scatter(hist_v, [d], c, mask=m)
                return 0
            lax.fori_loop(0, CHUNK16 // 16, hb, 0)
            pltpu.sync_copy(hist_v, grid_s.at[sid])

        plsc.subcore_barrier()

        @pl.when(on_core0)
        def _scan_perm():
            pltpu.sync_copy(grid_s, grid_v)

            def scan_body(g, carry):
                sl = pl.ds(g * 16, 16)
                tot = jnp.zeros((16,), jnp.int32)
                below = jnp.zeros((16,), jnp.int32)
                for t in range(16):
                    row = grid_v[t, sl]
                    tot = tot + row
                    below = below + jnp.where(t < sid, row, 0)
                incl = plsc.cumsum(tot)
                cur_v[sl] = carry + (incl - tot) + below
                return carry + jnp.sum(tot)

            lax.fori_loop(0, NBINS // 16, scan_body, jnp.int32(0))

            def pb(i, _):
                v = keys_v[pl.ds(i * 16, 16)]
                d = lax.shift_right_logical(v, shift) & (NBINS - 1)
                c, m = plsc.scan_count(d)
                base = plsc.load_gather(cur_v, [d])
                pos2_v[i >> 2, pl.ds((i & 3) * 16, 16)] = base + c - 1
                plsc.addupdate_scatter(cur_v, [d], c, mask=m)
                return 0

            lax.fori_loop(0, CHUNK16 // 16, pb, 0)
            nbuf = 4
            for j0 in range(0, CHUNK16 // 64, nbuf):
                cs = []
                for j in range(j0, min(j0 + nbuf, CHUNK16 // 64)):
                    cs.append(pltpu.async_copy(
                        keys_v.at[pl.ds(j * 64, 64)],
                        dst_k.at[pos2_v.at[j]], sem_k))
                    cs.append(pltpu.async_copy(
                        pays_v.at[pl.ds(j * 64, 64)],
                        dst_p.at[pos2_v.at[j]], sem_p))
                for c in cs:
                    c.wait()

        plsc.subcore_barrier()

    return pl.kernel(
        body,
        out_type=[jax.ShapeDtypeStruct((RP,), jnp.int32)] * 2,
        mesh=_mesh,
        scratch_types=[
            pltpu.VMEM((CHUNK16,), jnp.int32),
            pltpu.VMEM((CHUNK16,), jnp.int32),
            pltpu.VMEM((CHUNK16 // 64, 64), jnp.int32),
            pltpu.VMEM((NBINS,), jnp.int32),
            pltpu.VMEM((NBINS,), jnp.int32),
            pltpu.VMEM((16, NBINS), jnp.int32),
            pltpu.VMEM_SHARED((16, NBINS), jnp.int32),
            pltpu.SemaphoreType.DMA,
            pltpu.SemaphoreType.DMA,
        ],
        compiler_params=_sc_params,
    )


_pass0 = _make_pass(0, True)
_pass1 = _make_pass(11, False)
_pass2 = _make_pass(22, False)


def _radix_sort(key_pad):
    k, p = _pass0(key_pad)
    k, p = _pass1(k, p)
    k, p = _pass2(k, p)
    return k, p


def kernel(rel_logits, sub_logits, obj_logits, rel_pair_idx, bbox):
    n_box = bbox.shape[0]
    sub_ind = rel_pair_idx[:, 0]
    obj_ind = rel_pair_idx[:, 1]

    sum_s = jax.ops.segment_sum(sub_logits, sub_ind, num_segments=n_box)
    sum_o = jax.ops.segment_sum(obj_logits, obj_ind, num_segments=n_box)

    pad = jnp.full((RP - R,), N_BOX, jnp.int32)
    sub_pad = jnp.concatenate([sub_ind.astype(jnp.int32), pad])
    obj_pad = jnp.concatenate([obj_ind.astype(jnp.int32), pad])
    cnt_parts = _count_kernel(sub_pad, obj_pad)
    cnt = cnt_parts.sum(axis=0)[:N_BOX].astype(jnp.float32)

    refine_logits = (sum_s + sum_o) / jnp.maximum(cnt, 1.0)[:, None]
    obj_class_prob = jax.nn.softmax(refine_logits, axis=-1)
    obj_class_prob = obj_class_prob.at[:, 0].set(0.0)
    obj_scores = jnp.max(obj_class_prob[:, 1:], axis=1)
    obj_pred = jnp.argmax(obj_class_prob[:, 1:], axis=1) + 1

    obj_scores0 = obj_scores[sub_ind]
    obj_scores1 = obj_scores[obj_ind]
    rel_class_prob = jax.nn.softmax(rel_logits, axis=-1)
    rel_scores = jnp.max(rel_class_prob[:, 1:], axis=1)
    rel_class = jnp.argmax(rel_class_prob[:, 1:], axis=1) + 1
    triple_scores = rel_scores * obj_scores0 * obj_scores1

    bits = lax.bitcast_convert_type(triple_scores, jnp.int32)
    key = jnp.int32(0x7FFFFFFF) - bits
    key_pad = jnp.concatenate(
        [key, jnp.full((RP - R,), jnp.int32(0x7FFFFFFF))])

    _, perm_pad = _radix_sort(key_pad)
    perm = perm_pad[:R]

    rel_pair_sorted = rel_pair_idx[perm]
    rel_class_prob_sorted = rel_class_prob[perm]
    rel_labels = rel_class[perm]
    return (obj_pred, obj_scores, rel_pair_sorted, rel_class_prob_sorted,
            rel_labels)


# speedup vs baseline: 1.0049x; 1.0049x over previous
import functools

import jax
import jax.numpy as jnp
from jax import lax
from jax.experimental import pallas as pl
from jax.experimental.pallas import tpu as pltpu
from jax.experimental.pallas import tpu_sc as plsc

R = 50000
RP = 50176
N_BOX = 5000
NBINS_CNT = 5008
C_OBJ = 151
C_REL = 51
CHUNK32 = RP // 32
CHUNK16 = RP // 16
NBINS = 2048
LAST_OFF = R - CHUNK32

_mesh = plsc.VectorSubcoreMesh(core_axis_name="c", subcore_axis_name="s")
_sc_params = pltpu.CompilerParams(needs_layout_passes=False)


@functools.partial(
    pl.kernel,
    out_type=jax.ShapeDtypeStruct((32, NBINS_CNT), jnp.int32),
    mesh=_mesh,
    scratch_types=[
        pltpu.VMEM((CHUNK32,), jnp.int32),
        pltpu.VMEM((NBINS_CNT,), jnp.int32),
    ],
    compiler_params=_sc_params,
)
def _count_kernel(sub_hbm, obj_hbm, out_hbm, idx_v, hist_v):
    cid = lax.axis_index("c")
    sid = lax.axis_index("s")
    w = cid * 16 + sid
    o = w * CHUNK32

    def zero_body(i, _):
        hist_v[pl.ds(i * 16, 16)] = jnp.zeros((16,), jnp.int32)
        return 0

    lax.fori_loop(0, NBINS_CNT // 16, zero_body, 0)

    def count_chunk(src):
        pltpu.sync_copy(src.at[pl.ds(o, CHUNK32)], idx_v)

        def body(i, _):
            v = idx_v[pl.ds(i * 16, 16)]
            c, m = plsc.scan_count(v)
            plsc.addupdate_scatter(hist_v, [v], c, mask=m)
            return 0

        lax.fori_loop(0, CHUNK32 // 16, body, 0)

    count_chunk(sub_hbm)
    count_chunk(obj_hbm)
    pltpu.sync_copy(hist_v, out_hbm.at[w])


def _make_pass(shift, first):
    n_in = 1 if first else 2

    def body(*refs):
        if first:
            key_in, dst_k, dst_p = refs[:3]
            src_p = None
        else:
            key_in, src_p, dst_k, dst_p = refs[:4]
        (keys_v, pays_v, pos2_v, hist_v, cur_v, grid_v, grid_s,
         sem_k, sem_p) = refs[n_in + 2:]
        cid = lax.axis_index("c")
        sid = lax.axis_index("s")
        o = sid * CHUNK16
        on_core0 = cid == 0

        @pl.when(on_core0)
        def _hist():
            pltpu.sync_copy(key_in.at[pl.ds(o, CHUNK16)], keys_v)
            if first:
                def ib(i, _):
                    pays_v[pl.ds(i * 16, 16)] = (
                        o + i * 16 + lax.iota(jnp.int32, 16))
                    return 0
                lax.fori_loop(0, CHUNK16 // 16, ib, 0, unroll=4)
            else:
                pltpu.sync_copy(src_p.at[pl.ds(o, CHUNK16)], pays_v)

            def zb(i, _):
                hist_v[pl.ds(i * 16, 16)] = jnp.zeros((16,), jnp.int32)
                return 0
            lax.fori_loop(0, NBINS // 16, zb, 0, unroll=4)

            def hb(i, _):
                v = keys_v[pl.ds(i * 16, 16)]
                d = lax.shift_right_logical(v, shift) & (NBINS - 1)
                c, m = plsc.scan_count(d)
                plsc.addupdate_scatter(hist_v, [d], c, mask=m)
                return 0
            lax.fori_loop(0, CHUNK16 // 16, hb, 0, unroll=4)
            pltpu.sync_copy(hist_v, grid_s.at[sid])

        plsc.subcore_barrier()

        @pl.when(on_core0)
        def _scan_perm():
            pltpu.sync_copy(grid_s, grid_v)

            def scan_body(g, carry):
                sl = pl.ds(g * 16, 16)
                tot = jnp.zeros((16,), jnp.int32)
                below = jnp.zeros((16,), jnp.int32)
                for t in range(16):
                    row = grid_v[t, sl]
                    tot = tot + row
                    below = below + jnp.where(t < sid, row, 0)
                incl = plsc.cumsum(tot)
                cur_v[sl] = carry + (incl - tot) + below
                return carry + jnp.sum(tot)

            lax.fori_loop(0, NBINS // 16, scan_body, jnp.int32(0), unroll=2)

            def pb(i, _):
                v = keys_v[pl.ds(i * 16, 16)]
                d = lax.shift_right_logical(v, shift) & (NBINS - 1)
                c, m = plsc.scan_count(d)
                base = plsc.load_gather(cur_v, [d])
                pos2_v[i >> 2, pl.ds((i & 3) * 16, 16)] = base + c - 1
                plsc.addupdate_scatter(cur_v, [d], c, mask=m)
                return 0

            lax.fori_loop(0, CHUNK16 // 16, pb, 0, unroll=4)
            nbuf = 12
            for j0 in range(0, CHUNK16 // 64, nbuf):
                cs = []
                for j in range(j0, min(j0 + nbuf, CHUNK16 // 64)):
                    cs.append(pltpu.async_copy(
                        keys_v.at[pl.ds(j * 64, 64)],
                        dst_k.at[pos2_v.at[j]], sem_k))
                    cs.append(pltpu.async_copy(
                        pays_v.at[pl.ds(j * 64, 64)],
                        dst_p.at[pos2_v.at[j]], sem_p))
                for c in cs:
                    c.wait()

        plsc.subcore_barrier()

    return pl.kernel(
        body,
        out_type=[jax.ShapeDtypeStruct((RP,), jnp.int32)] * 2,
        mesh=_mesh,
        scratch_types=[
            pltpu.VMEM((CHUNK16,), jnp.int32),
            pltpu.VMEM((CHUNK16,), jnp.int32),
            pltpu.VMEM((CHUNK16 // 64, 64), jnp.int32),
            pltpu.VMEM((NBINS,), jnp.int32),
            pltpu.VMEM((NBINS,), jnp.int32),
            pltpu.VMEM((16, NBINS), jnp.int32),
            pltpu.VMEM_SHARED((16, NBINS), jnp.int32),
            pltpu.SemaphoreType.DMA,
            pltpu.SemaphoreType.DMA,
        ],
        compiler_params=_sc_params,
    )


_pass0 = _make_pass(0, True)
_pass1 = _make_pass(11, False)
_pass2 = _make_pass(22, False)


def _radix_sort(key_pad):
    k, p = _pass0(key_pad)
    k, p = _pass1(k, p)
    k, p = _pass2(k, p)
    return k, p


def kernel(rel_logits, sub_logits, obj_logits, rel_pair_idx, bbox):
    n_box = bbox.shape[0]
    sub_ind = rel_pair_idx[:, 0]
    obj_ind = rel_pair_idx[:, 1]

    sum_s = jax.ops.segment_sum(sub_logits, sub_ind, num_segments=n_box)
    sum_o = jax.ops.segment_sum(obj_logits, obj_ind, num_segments=n_box)

    pad = jnp.full((RP - R,), N_BOX, jnp.int32)
    sub_pad = jnp.concatenate([sub_ind.astype(jnp.int32), pad])
    obj_pad = jnp.concatenate([obj_ind.astype(jnp.int32), pad])
    cnt_parts = _count_kernel(sub_pad, obj_pad)
    cnt = cnt_parts.sum(axis=0)[:N_BOX].astype(jnp.float32)

    refine_logits = (sum_s + sum_o) / jnp.maximum(cnt, 1.0)[:, None]
    obj_class_prob = jax.nn.softmax(refine_logits, axis=-1)
    obj_class_prob = obj_class_prob.at[:, 0].set(0.0)
    obj_scores = jnp.max(obj_class_prob[:, 1:], axis=1)
    obj_pred = jnp.argmax(obj_class_prob[:, 1:], axis=1) + 1

    obj_scores0 = obj_scores[sub_ind]
    obj_scores1 = obj_scores[obj_ind]
    rel_class_prob = jax.nn.softmax(rel_logits, axis=-1)
    rel_scores = jnp.max(rel_class_prob[:, 1:], axis=1)
    rel_class = jnp.argmax(rel_class_prob[:, 1:], axis=1) + 1
    triple_scores = rel_scores * obj_scores0 * obj_scores1

    bits = lax.bitcast_convert_type(triple_scores, jnp.int32)
    key = jnp.int32(0x7FFFFFFF) - bits
    key_pad = jnp.concatenate(
        [key, jnp.full((RP - R,), jnp.int32(0x7FFFFFFF))])

    _, perm_pad = _radix_sort(key_pad)
    perm = perm_pad[:R]

    rel_pair_sorted = rel_pair_idx[perm]
    rel_class_prob_sorted = rel_class_prob[perm]
    rel_labels = rel_class[perm]
    return (obj_pred, obj_scores, rel_pair_sorted, rel_class_prob_sorted,
            rel_labels)
